# baseline (speedup 1.0000x reference)
import functools

import jax
import jax.numpy as jnp
from jax import lax
from jax.experimental import pallas as pl
from jax.experimental.pallas import tpu as pltpu
from jax.experimental.pallas import tpu_sc as plsc

_S = 128
_L = 16
_BLK = 128
_HROWS = 129
_G = 2


def _build(R, NB):
    NW = 32
    rays_per_w = R // NW
    n_blk = rays_per_w // _BLK
    NWT = NB - 1
    mesh = plsc.VectorSubcoreMesh(core_axis_name="c", subcore_axis_name="s")

    @functools.partial(
        pl.kernel,
        mesh=mesh,
        compiler_params=pltpu.CompilerParams(needs_layout_passes=False),
        out_type=jax.ShapeDtypeStruct((R * _S,), jnp.float32),
        scratch_types=[
            pltpu.VMEM((_BLK * NWT,), jnp.float32),
            pltpu.VMEM((_BLK * NB,), jnp.float32),
            pltpu.VMEM((_BLK * _S,), jnp.float32),
            pltpu.VMEM((_G * NWT * _L,), jnp.float32),
            pltpu.VMEM((_G * NB * _L,), jnp.float32),
            pltpu.VMEM((_G * _HROWS * _L,), jnp.int32),
        ],
    )
    def k(bins_hbm, w_hbm, out_hbm, wbuf, bbuf, obuf, cbuf, cdf2, hist):
        wid = lax.axis_index("c") * 16 + lax.axis_index("s")
        lane = lax.iota(jnp.int32, _L)
        ones_i = jnp.ones((_L,), jnp.int32)
        zeros_f = jnp.zeros((_L,), jnp.float32)
        zeros_i = jnp.zeros((_L,), jnp.int32)
        u0 = jnp.full((_L,), 1.0 / (2.0 * _S), jnp.float32)
        du = 1.0 / _S
        coff = [s * NWT * _L for s in range(_G)]
        doff = [s * NB * _L for s in range(_G)]
        hoff = [s * _HROWS * _L for s in range(_G)]
        lane_d = [lane + s * NB * _L for s in range(_G)]
        lane_h = [lane + s * _HROWS * _L for s in range(_G)]

        def block_body(b, _):
            base = (wid * n_blk + b) * _BLK
            pltpu.sync_copy(w_hbm.at[pl.ds(base * NWT, _BLK * NWT)], wbuf)
            pltpu.sync_copy(bins_hbm.at[pl.ds(base * NB, _BLK * NB)], bbuf)

            def group_body(g, _):
                rows_w = [((_G * g + s) * _L + lane) * NWT for s in range(_G)]
                rows_b = [((_G * g + s) * _L + lane) * NB for s in range(_G)]
                rows_o = [((_G * g + s) * _L + lane) * _S for s in range(_G)]

                @plsc.parallel_loop(0, NWT, carry=(zeros_f,) * _G, unroll=4)
                def p1(i, accs):
                    out = []
                    for s in range(_G):
                        w = plsc.load_gather(wbuf, [rows_w[s] + i]) + 1e-5
                        a = accs[s] + w
                        cbuf[pl.ds(coff[s] + i * _L, _L)] = a
                        hist[pl.ds(hoff[s] + i * _L, _L)] = zeros_i
                        out.append(a)
                    return tuple(out)

                invs = [1.0 / t for t in p1]

                for s in range(_G):
                    cdf2[pl.ds(doff[s], _L)] = zeros_f
                    hist[pl.ds(hoff[s] + NWT * _L, _L)] = zeros_i
                    hist[pl.ds(hoff[s] + (NWT + 1) * _L, _L)] = zeros_i
                    hist[pl.ds(hoff[s] + (NWT + 2) * _L, _L)] = zeros_i

                @plsc.parallel_loop(0, NWT, unroll=4)
                def p2(i):
                    for s in range(_G):
                        cv = cbuf[pl.ds(coff[s] + i * _L, _L)] * invs[s]
                        cdf2[pl.ds(doff[s] + (i + 1) * _L, _L)] = cv
                        y = cv * float(_S) - 0.5
                        t0 = y.astype(jnp.int32)
                        m = t0 + (t0.astype(jnp.float32) < y).astype(jnp.int32)
                        plsc.addupdate_scatter(
                            hist, [m * _L + lane_h[s]], ones_i)

                @plsc.parallel_loop(0, _S, carry=(zeros_i,) * _G + (u0,),
                                    unroll=4)
                def p3(j, carry):
                    accs, u = carry[:_G], carry[_G]
                    out = []
                    for s in range(_G):
                        h = hist[pl.ds(hoff[s] + j * _L, _L)]
                        bl = accs[s] + h
                        ab = jnp.minimum(bl + 1, NB - 1)
                        c0 = plsc.load_gather(cdf2, [bl * _L + lane_d[s]])
                        c1 = plsc.load_gather(cdf2, [ab * _L + lane_d[s]])
                        b0 = plsc.load_gather(bbuf, [rows_b[s] + bl])
                        b1 = plsc.load_gather(bbuf, [rows_b[s] + ab])
                        dn = c1 - c0
                        dn = jnp.where(dn < 1e-5, 1.0, dn)
                        t = (u - c0) / dn
                        sv = b0 + t * (b1 - b0)
                        plsc.store_scatter(obuf, [rows_o[s] + j], sv)
                        out.append(bl)
                    return tuple(out) + (u + du,)

                del p3
                return 0

            lax.fori_loop(0, _BLK // (_G * _L), group_body, 0)
            pltpu.sync_copy(obuf, out_hbm.at[pl.ds(base * _S, _BLK * _S)])
            return 0

        lax.fori_loop(0, n_blk, block_body, 0)

    return k


def kernel(bins, weights, n_samples):
    R, NB = bins.shape
    out = _build(R, NB)(bins.reshape(-1), weights.reshape(-1))
    return out.reshape(R, _S)


# speedup vs baseline: 1.0674x; 1.0674x over previous
import functools

import jax
import jax.numpy as jnp
from jax import lax
from jax.experimental import pallas as pl
from jax.experimental.pallas import tpu as pltpu
from jax.experimental.pallas import tpu_sc as plsc

_S = 128
_L = 16
_BLK = 128
_HROWS = 129


def _build(R, NB):
    NW = 32
    rays_per_w = R // NW
    n_blk = rays_per_w // _BLK
    NWT = NB - 1
    mesh = plsc.VectorSubcoreMesh(core_axis_name="c", subcore_axis_name="s")

    @functools.partial(
        pl.kernel,
        mesh=mesh,
        compiler_params=pltpu.CompilerParams(needs_layout_passes=False),
        out_type=jax.ShapeDtypeStruct((R * _S,), jnp.float32),
        scratch_types=[
            pltpu.VMEM((_BLK * NWT,), jnp.float32),
            pltpu.VMEM((_BLK * NB,), jnp.float32),
            pltpu.VMEM((_BLK * _S,), jnp.float32),
            pltpu.VMEM((NB * 2 * _L,), jnp.float32),
            pltpu.VMEM((_HROWS * _L,), jnp.int32),
        ],
    )
    def k(bins_hbm, w_hbm, out_hbm, wbuf, bbuf, obuf, pq, hist):
        wid = lax.axis_index("c") * 16 + lax.axis_index("s")
        lane = lax.iota(jnp.int32, _L)
        ones_i = jnp.ones((_L,), jnp.int32)
        zeros_f = jnp.zeros((_L,), jnp.float32)
        zeros_i = jnp.zeros((_L,), jnp.int32)
        u0 = jnp.full((_L,), 1.0 / (2.0 * _S), jnp.float32)
        du = 1.0 / _S

        def block_body(b, _):
            base = (wid * n_blk + b) * _BLK
            pltpu.sync_copy(w_hbm.at[pl.ds(base * NWT, _BLK * NWT)], wbuf)
            pltpu.sync_copy(bins_hbm.at[pl.ds(base * NB, _BLK * NB)], bbuf)

            def group_body(g, _):
                rows_w = (g * _L + lane) * NWT
                rows_b = (g * _L + lane) * NB
                rows_o = (g * _L + lane) * _S

                b0 = plsc.load_gather(bbuf, [rows_b])

                @plsc.parallel_loop(0, NWT, carry=(zeros_f, b0), unroll=6)
                def p1(i, carry):
                    acc, pb = carry
                    w = plsc.load_gather(wbuf, [rows_w + i])
                    nb = plsc.load_gather(bbuf, [rows_b + (i + 1)])
                    pu = w + 1e-5
                    qu = (nb - pb) / pu
                    pq[pl.ds(i * 2 * _L, _L)] = pb - acc * qu
                    pq[pl.ds(i * 2 * _L + _L, _L)] = qu
                    hist[pl.ds(i * _L, _L)] = zeros_i
                    return acc + pu, nb

                total, blast = p1
                s128 = float(_S) / total

                pq[pl.ds(NWT * 2 * _L, _L)] = blast
                pq[pl.ds(NWT * 2 * _L + _L, _L)] = zeros_f
                hist[pl.ds(NWT * _L, _L)] = zeros_i
                hist[pl.ds((NWT + 1) * _L, _L)] = zeros_i
                hist[pl.ds((NWT + 2) * _L, _L)] = zeros_i

                @plsc.parallel_loop(0, NWT, carry=zeros_f, unroll=6)
                def p2(i, acc):
                    acc = acc + (plsc.load_gather(wbuf, [rows_w + i]) + 1e-5)
                    y = acc * s128 - 0.5
                    t0 = y.astype(jnp.int32)
                    m = t0 + (t0.astype(jnp.float32) < y).astype(jnp.int32)
                    plsc.addupdate_scatter(hist, [m * _L + lane], ones_i)
                    return acc

                del p2

                @plsc.parallel_loop(0, _S, carry=(zeros_i, u0), unroll=8)
                def p3(j, carry):
                    acc, u = carry
                    bl = acc + hist[pl.ds(j * _L, _L)]
                    idx = bl * (2 * _L) + lane
                    pv = plsc.load_gather(pq, [idx])
                    qv = plsc.load_gather(pq, [idx + _L])
                    sv = pv + (u * total) * qv
                    plsc.store_scatter(obuf, [rows_o + j], sv)
                    return bl, u + du

                del p3
                return 0

            lax.fori_loop(0, _BLK // _L, group_body, 0)
            pltpu.sync_copy(obuf, out_hbm.at[pl.ds(base * _S, _BLK * _S)])
            return 0

        lax.fori_loop(0, n_blk, block_body, 0)

    return k


def kernel(bins, weights, n_samples):
    R, NB = bins.shape
    out = _build(R, NB)(bins.reshape(-1), weights.reshape(-1))
    return out.reshape(R, _S)


# speedup vs baseline: 1.1031x; 1.0335x over previous
import functools

import jax
import jax.numpy as jnp
from jax import lax
from jax.experimental import pallas as pl
from jax.experimental.pallas import tpu as pltpu
from jax.experimental.pallas import tpu_sc as plsc

_S = 128
_L = 16
_BLK = 128
_HROWS = 129


def _build(R, NB):
    NW = 32
    rays_per_w = R // NW
    n_blk = rays_per_w // _BLK
    NWT = NB - 1
    mesh = plsc.VectorSubcoreMesh(core_axis_name="c", subcore_axis_name="s")

    @functools.partial(
        pl.kernel,
        mesh=mesh,
        compiler_params=pltpu.CompilerParams(needs_layout_passes=False),
        out_type=jax.ShapeDtypeStruct((R * _S,), jnp.float32),
        scratch_types=[
            pltpu.VMEM((_BLK * NWT,), jnp.float32),
            pltpu.VMEM((_BLK * NB,), jnp.float32),
            pltpu.VMEM((_BLK * _S,), jnp.float32),
            pltpu.VMEM((NB * 2 * _L,), jnp.float32),
            pltpu.VMEM((_HROWS * _L,), jnp.int32),
        ],
    )
    def k(bins_hbm, w_hbm, out_hbm, wbuf, bbuf, obuf, pq, hist):
        wid = lax.axis_index("c") * 16 + lax.axis_index("s")
        lane = lax.iota(jnp.int32, _L)
        ones_i = jnp.ones((_L,), jnp.int32)
        zeros_f = jnp.zeros((_L,), jnp.float32)
        zeros_i = jnp.zeros((_L,), jnp.int32)
        u0 = jnp.full((_L,), 1.0 / (2.0 * _S), jnp.float32)
        du = 1.0 / _S

        def block_body(b, _):
            base = (wid * n_blk + b) * _BLK
            pltpu.sync_copy(w_hbm.at[pl.ds(base * NWT, _BLK * NWT)], wbuf)
            pltpu.sync_copy(bins_hbm.at[pl.ds(base * NB, _BLK * NB)], bbuf)

            def group_body(g, _):
                rows_w = (g * _L + lane) * NWT
                rows_b = (g * _L + lane) * NB
                rows_o = (g * _L + lane) * _S

                b0 = plsc.load_gather(bbuf, [rows_b])

                @plsc.parallel_loop(0, NWT, carry=(zeros_f, b0), unroll=9)
                def p1(i, carry):
                    acc, pb = carry
                    w = plsc.load_gather(wbuf, [rows_w + i])
                    nb = plsc.load_gather(bbuf, [rows_b + (i + 1)])
                    pu = w + 1e-5
                    qu = (nb - pb) / pu
                    pq[pl.ds(i * 2 * _L, _L)] = pb - acc * qu
                    pq[pl.ds(i * 2 * _L + _L, _L)] = qu
                    hist[pl.ds(i * _L, _L)] = zeros_i
                    return acc + pu, nb

                total, blast = p1
                s128 = float(_S) / total

                pq[pl.ds(NWT * 2 * _L, _L)] = blast
                pq[pl.ds(NWT * 2 * _L + _L, _L)] = zeros_f
                hist[pl.ds(NWT * _L, _L)] = zeros_i
                hist[pl.ds((NWT + 1) * _L, _L)] = zeros_i
                hist[pl.ds((NWT + 2) * _L, _L)] = zeros_i

                @plsc.parallel_loop(0, NWT, carry=zeros_f, unroll=9)
                def p2(i, acc):
                    acc = acc + (plsc.load_gather(wbuf, [rows_w + i]) + 1e-5)
                    m = _S - ((_S + 0.5) - acc * s128).astype(jnp.int32)
                    plsc.addupdate_scatter(hist, [m * _L + lane], ones_i)
                    return acc

                del p2

                @plsc.parallel_loop(0, _S, carry=(zeros_i, u0), unroll=16)
                def p3(j, carry):
                    acc, u = carry
                    bl = acc + hist[pl.ds(j * _L, _L)]
                    idx = bl * (2 * _L) + lane
                    pv = plsc.load_gather(pq, [idx])
                    qv = plsc.load_gather(pq, [idx + _L])
                    sv = pv + (u * total) * qv
                    plsc.store_scatter(obuf, [rows_o + j], sv)
                    return bl, u + du

                del p3
                return 0

            lax.fori_loop(0, _BLK // _L, group_body, 0)
            pltpu.sync_copy(obuf, out_hbm.at[pl.ds(base * _S, _BLK * _S)])
            return 0

        lax.fori_loop(0, n_blk, block_body, 0)

    return k


def kernel(bins, weights, n_samples):
    R, NB = bins.shape
    out = _build(R, NB)(bins.reshape(-1), weights.reshape(-1))
    return out.reshape(R, _S)


# speedup vs baseline: 1.2523x; 1.1352x over previous
import functools

import jax
import jax.numpy as jnp
from jax import lax
from jax.experimental import pallas as pl
from jax.experimental.pallas import tpu as pltpu
from jax.experimental.pallas import tpu_sc as plsc

_S = 128
_L = 16
_BLK = 128
_HROWS = 129


def _build(R, NB):
    NW = 32
    rays_per_w = R // NW
    n_blk = rays_per_w // _BLK
    NWT = NB - 1
    mesh = plsc.VectorSubcoreMesh(core_axis_name="c", subcore_axis_name="s")

    @functools.partial(
        pl.kernel,
        mesh=mesh,
        compiler_params=pltpu.CompilerParams(needs_layout_passes=False),
        out_type=jax.ShapeDtypeStruct((R * _S,), jnp.float32),
        scratch_types=[
            pltpu.VMEM((_BLK * NWT,), jnp.float32),
            pltpu.VMEM((_BLK * NB,), jnp.float32),
            pltpu.VMEM((_BLK * NWT,), jnp.float32),
            pltpu.VMEM((_BLK * NB,), jnp.float32),
            pltpu.VMEM((_BLK * _S,), jnp.float32),
            pltpu.VMEM((NB * 2 * _L,), jnp.float32),
            pltpu.VMEM((_HROWS * _L,), jnp.int32),
            pltpu.SemaphoreType.DMA,
            pltpu.SemaphoreType.DMA,
        ],
    )
    def k(bins_hbm, w_hbm, out_hbm, wbuf0, bbuf0, wbuf1, bbuf1, obuf, pq,
          hist, sem0, sem1):
        wid = lax.axis_index("c") * 16 + lax.axis_index("s")
        lane = lax.iota(jnp.int32, _L)
        ones_i = jnp.ones((_L,), jnp.int32)
        zeros_f = jnp.zeros((_L,), jnp.float32)
        zeros_i = jnp.zeros((_L,), jnp.int32)
        u0 = jnp.full((_L,), 1.0 / (2.0 * _S), jnp.float32)
        du = 1.0 / _S

        def start_in(b, wb, bb, sem):
            base = (wid * n_blk + b) * _BLK
            pltpu.async_copy(w_hbm.at[pl.ds(base * NWT, _BLK * NWT)], wb, sem)
            pltpu.async_copy(bins_hbm.at[pl.ds(base * NB, _BLK * NB)], bb, sem)

        def wait_in(wb, bb, sem):
            pltpu.make_async_copy(
                w_hbm.at[pl.ds(0, _BLK * NWT)], wb, sem).wait()
            pltpu.make_async_copy(
                bins_hbm.at[pl.ds(0, _BLK * NB)], bb, sem).wait()

        def compute_block(b, wbuf, bbuf):
            def group_body(g, _):
                rows_w = (g * _L + lane) * NWT
                rows_b = (g * _L + lane) * NB
                rows_o = (g * _L + lane) * _S

                b0 = plsc.load_gather(bbuf, [rows_b])

                @plsc.parallel_loop(0, NWT, carry=(zeros_f, b0), unroll=9)
                def p1(i, carry):
                    acc, pb = carry
                    w = plsc.load_gather(wbuf, [rows_w + i])
                    nb = plsc.load_gather(bbuf, [rows_b + (i + 1)])
                    pu = w + 1e-5
                    qu = (nb - pb) / pu
                    pq[pl.ds(i * 2 * _L, _L)] = pb - acc * qu
                    pq[pl.ds(i * 2 * _L + _L, _L)] = qu
                    hist[pl.ds(i * _L, _L)] = zeros_i
                    return acc + pu, nb

                total, blast = p1
                s128 = float(_S) / total

                pq[pl.ds(NWT * 2 * _L, _L)] = blast
                pq[pl.ds(NWT * 2 * _L + _L, _L)] = zeros_f
                hist[pl.ds(NWT * _L, _L)] = zeros_i
                hist[pl.ds((NWT + 1) * _L, _L)] = zeros_i
                hist[pl.ds((NWT + 2) * _L, _L)] = zeros_i

                @plsc.parallel_loop(0, NWT, carry=zeros_f, unroll=9)
                def p2(i, acc):
                    acc = acc + (plsc.load_gather(wbuf, [rows_w + i]) + 1e-5)
                    m = _S - ((_S + 0.5) - acc * s128).astype(jnp.int32)
                    plsc.addupdate_scatter(hist, [m * _L + lane], ones_i)
                    return acc

                del p2

                @plsc.parallel_loop(0, _S, carry=(zeros_i, u0), unroll=16)
                def p3(j, carry):
                    acc, u = carry
                    bl = acc + hist[pl.ds(j * _L, _L)]
                    idx = bl * (2 * _L) + lane
                    pv = plsc.load_gather(pq, [idx])
                    qv = plsc.load_gather(pq, [idx + _L])
                    sv = pv + (u * total) * qv
                    plsc.store_scatter(obuf, [rows_o + j], sv)
                    return bl, u + du

                del p3
                return 0

            lax.fori_loop(0, _BLK // _L, group_body, 0)
            base = (wid * n_blk + b) * _BLK
            pltpu.sync_copy(obuf, out_hbm.at[pl.ds(base * _S, _BLK * _S)])

        start_in(0, wbuf0, bbuf0, sem0)

        def pair_body(t, _):
            b = 2 * t
            wait_in(wbuf0, bbuf0, sem0)
            start_in(b + 1, wbuf1, bbuf1, sem1)
            compute_block(b, wbuf0, bbuf0)
            wait_in(wbuf1, bbuf1, sem1)

            @pl.when(t < n_blk // 2 - 1)
            def _():
                start_in(b + 2, wbuf0, bbuf0, sem0)

            compute_block(b + 1, wbuf1, bbuf1)
            return 0

        lax.fori_loop(0, n_blk // 2, pair_body, 0)

    return k


def kernel(bins, weights, n_samples):
    R, NB = bins.shape
    out = _build(R, NB)(bins.reshape(-1), weights.reshape(-1))
    return out.reshape(R, _S)


# speedup vs baseline: 1.2929x; 1.0325x over previous
import functools

import jax
import jax.numpy as jnp
from jax import lax
from jax.experimental import pallas as pl
from jax.experimental.pallas import tpu as pltpu
from jax.experimental.pallas import tpu_sc as plsc

_S = 128
_L = 16
_BLK = 128
_HROWS = 129


def _build(R, NB):
    NW = 32
    rays_per_w = R // NW
    n_blk = rays_per_w // _BLK
    NWT = NB - 1
    mesh = plsc.VectorSubcoreMesh(core_axis_name="c", subcore_axis_name="s")

    @functools.partial(
        pl.kernel,
        mesh=mesh,
        compiler_params=pltpu.CompilerParams(needs_layout_passes=False),
        out_type=jax.ShapeDtypeStruct((R * _S,), jnp.float32),
        scratch_types=[
            pltpu.VMEM((_BLK * NWT,), jnp.float32),
            pltpu.VMEM((_BLK * NB,), jnp.float32),
            pltpu.VMEM((_BLK * NWT,), jnp.float32),
            pltpu.VMEM((_BLK * NB,), jnp.float32),
            pltpu.VMEM((_BLK * _S,), jnp.float32),
            pltpu.VMEM((_BLK * _S,), jnp.float32),
            pltpu.VMEM((NB * 2 * _L,), jnp.float32),
            pltpu.VMEM((_HROWS * _L,), jnp.int32),
            pltpu.SemaphoreType.DMA,
            pltpu.SemaphoreType.DMA,
            pltpu.SemaphoreType.DMA,
            pltpu.SemaphoreType.DMA,
        ],
    )
    def k(bins_hbm, w_hbm, out_hbm, wbuf0, bbuf0, wbuf1, bbuf1, obuf0, obuf1,
          pq, hist, sem0, sem1, semo0, semo1):
        wid = lax.axis_index("c") * 16 + lax.axis_index("s")
        lane = lax.iota(jnp.int32, _L)
        ones_i = jnp.ones((_L,), jnp.int32)
        zeros_f = jnp.zeros((_L,), jnp.float32)
        zeros_i = jnp.zeros((_L,), jnp.int32)
        u0 = jnp.full((_L,), 1.0 / (2.0 * _S), jnp.float32)
        du = 1.0 / _S

        def start_in(b, wb, bb, sem):
            base = (wid * n_blk + b) * _BLK
            pltpu.async_copy(w_hbm.at[pl.ds(base * NWT, _BLK * NWT)], wb, sem)
            pltpu.async_copy(bins_hbm.at[pl.ds(base * NB, _BLK * NB)], bb, sem)

        def wait_in(wb, bb, sem):
            pltpu.make_async_copy(
                w_hbm.at[pl.ds(0, _BLK * NWT)], wb, sem).wait()
            pltpu.make_async_copy(
                bins_hbm.at[pl.ds(0, _BLK * NB)], bb, sem).wait()

        def wait_out(ob, semo):
            pltpu.make_async_copy(
                ob, out_hbm.at[pl.ds(0, _BLK * _S)], semo).wait()

        def compute_block(b, wbuf, bbuf, obuf, semo, wait_prev):
            @pl.when(wait_prev)
            def _():
                wait_out(obuf, semo)

            def group_body(g, _):
                rows_w = (g * _L + lane) * NWT
                rows_b = (g * _L + lane) * NB
                rows_o = (g * _L + lane) * _S

                b0 = plsc.load_gather(bbuf, [rows_b])

                @plsc.parallel_loop(0, NWT, carry=(zeros_f, b0), unroll=9)
                def p1(i, carry):
                    acc, pb = carry
                    w = plsc.load_gather(wbuf, [rows_w + i])
                    nb = plsc.load_gather(bbuf, [rows_b + (i + 1)])
                    pu = w + 1e-5
                    qu = (nb - pb) / pu
                    pq[pl.ds(i * 2 * _L, _L)] = pb - acc * qu
                    pq[pl.ds(i * 2 * _L + _L, _L)] = qu
                    hist[pl.ds(i * _L, _L)] = zeros_i
                    return acc + pu, nb

                total, blast = p1
                s128 = float(_S) / total

                pq[pl.ds(NWT * 2 * _L, _L)] = blast
                pq[pl.ds(NWT * 2 * _L + _L, _L)] = zeros_f
                hist[pl.ds(NWT * _L, _L)] = zeros_i
                hist[pl.ds((NWT + 1) * _L, _L)] = zeros_i
                hist[pl.ds((NWT + 2) * _L, _L)] = zeros_i

                @plsc.parallel_loop(0, NWT, carry=zeros_f, unroll=9)
                def p2(i, acc):
                    acc = acc + (plsc.load_gather(wbuf, [rows_w + i]) + 1e-5)
                    m = _S - ((_S + 0.5) - acc * s128).astype(jnp.int32)
                    plsc.addupdate_scatter(hist, [m * _L + lane], ones_i)
                    return acc

                del p2

                @plsc.parallel_loop(0, _S, carry=(zeros_i, u0), unroll=16)
                def p3(j, carry):
                    acc, u = carry
                    bl = acc + hist[pl.ds(j * _L, _L)]
                    idx = bl * (2 * _L) + lane
                    pv = plsc.load_gather(pq, [idx])
                    qv = plsc.load_gather(pq, [idx + _L])
                    sv = pv + (u * total) * qv
                    plsc.store_scatter(obuf, [rows_o + j], sv)
                    return bl, u + du

                del p3
                return 0

            lax.fori_loop(0, _BLK // _L, group_body, 0)
            base = (wid * n_blk + b) * _BLK
            pltpu.async_copy(
                obuf, out_hbm.at[pl.ds(base * _S, _BLK * _S)], semo)

        start_in(0, wbuf0, bbuf0, sem0)

        def pair_body(t, _):
            b = 2 * t
            wait_in(wbuf0, bbuf0, sem0)
            start_in(b + 1, wbuf1, bbuf1, sem1)
            compute_block(b, wbuf0, bbuf0, obuf0, semo0, t > 0)
            wait_in(wbuf1, bbuf1, sem1)

            @pl.when(t < n_blk // 2 - 1)
            def _():
                start_in(b + 2, wbuf0, bbuf0, sem0)

            compute_block(b + 1, wbuf1, bbuf1, obuf1, semo1, t > 0)
            return 0

        lax.fori_loop(0, n_blk // 2, pair_body, 0)
        wait_out(obuf0, semo0)
        wait_out(obuf1, semo1)

    return k


def kernel(bins, weights, n_samples):
    R, NB = bins.shape
    out = _build(R, NB)(bins.reshape(-1), weights.reshape(-1))
    return out.reshape(R, _S)
